# baseline (speedup 1.0000x reference)
import jax
import jax.numpy as jnp
from jax import lax
from jax.experimental import pallas as pl
from jax.experimental.pallas import tpu as pltpu

NEG2POS_RATIO = 3


def _ohem_body(x_ref, t16_ref, out_ref):
    B, N2 = x_ref.shape
    N = N2 // 2
    x = x_ref[...]
    te = t16_ref[...].astype(jnp.int32)

    xr = pltpu.roll(x, N2 - 1, 1)
    d = xr - x
    y = jnp.maximum(d, 0.0) + jnp.log1p(jnp.exp(-jnp.abs(d)))
    w = y - d

    lane = lax.broadcasted_iota(jnp.int32, (B, N2), 1)
    even = (lane & 1) == 0
    pos_e = even & (te == 1)
    neg_e = even & (te == 0)

    num_pos = jnp.sum(te, axis=1, keepdims=True)
    pos_sum = jnp.sum(jnp.where(pos_e, w, 0.0))
    cls_loss = jnp.where(neg_e, y, 0.0)
    u = lax.bitcast_convert_type(cls_loss, jnp.int32)
    k = jnp.clip(NEG2POS_RATIO * num_pos, 1, N - 1)

    cpos = jnp.sum((u > 0).astype(jnp.int32), axis=1, keepdims=True)
    shortcut = jnp.all(k >= cpos)

    def fast(_):
        return jnp.sum(cls_loss)

    def slow(_):
        def step(i, T):
            bit = 30 - i
            cand = T | lax.shift_left(jnp.int32(1), bit)
            cnt = jnp.sum((u >= cand).astype(jnp.int32), axis=1, keepdims=True)
            return jnp.where(cnt >= k, cand, T)

        T = lax.fori_loop(0, 31, step, jnp.zeros((B, 1), jnp.int32))
        tval = lax.bitcast_convert_type(T, jnp.float32)
        gt = u > T
        c_gt = jnp.sum(gt.astype(jnp.int32), axis=1, keepdims=True)
        sum_gt = jnp.sum(jnp.where(gt, cls_loss, 0.0), axis=1, keepdims=True)
        return jnp.sum(sum_gt + (k - c_gt).astype(jnp.float32) * tval)

    neg_sum = lax.cond(shortcut, fast, slow, None)

    total_pos = jnp.maximum(jnp.sum(num_pos).astype(jnp.float32), 1.0)
    res = (pos_sum + neg_sum) / total_pos
    out_ref[...] = jnp.reshape(res, (1, 1))


def kernel(cls_preds, cls_targets):
    B, N, _ = cls_preds.shape
    x = jnp.reshape(cls_preds, (B, 2 * N))
    t16 = jnp.reshape(
        lax.bitcast_convert_type(cls_targets.astype(jnp.int32), jnp.int16),
        (B, 2 * N))
    out = pl.pallas_call(
        _ohem_body,
        out_shape=jax.ShapeDtypeStruct((1, 1), jnp.float32),
    )(x, t16)
    return out[0, 0]
